# baseline (speedup 1.0000x reference)
import functools

import jax
import jax.numpy as jnp
from jax import lax
from jax.experimental import pallas as pl
from jax.experimental.pallas import tpu as pltpu
from jax.experimental.pallas import tpu_sc as plsc

N_ITEMS = 4096
N_SESSIONS = 2048
N = N_SESSIONS + N_ITEMS


def _sc_gather(table, idx):
    info = plsc.get_sparse_core_info()
    nc, ns = info.num_cores, info.num_subcores
    nw = nc * ns
    b, d = idx.shape[0], table.shape[1]
    b_per_w = b // nw
    mesh = plsc.VectorSubcoreMesh(core_axis_name="c", subcore_axis_name="s")

    @functools.partial(
        pl.kernel,
        mesh=mesh,
        out_type=jax.ShapeDtypeStruct((b, d), jnp.float32),
        scratch_types=[
            pltpu.VMEM((b_per_w,), jnp.int32),
            pltpu.VMEM((b_per_w, d), jnp.float32),
            pltpu.SemaphoreType.DMA,
        ],
    )
    def gather_kernel(table_hbm, idx_hbm, out_hbm, idx_v, rows_v, sem):
        wid = lax.axis_index("s") * nc + lax.axis_index("c")
        base = wid * b_per_w
        pltpu.sync_copy(idx_hbm.at[pl.ds(base, b_per_w)], idx_v)
        pltpu.async_copy(table_hbm.at[idx_v], rows_v, sem).wait()
        pltpu.sync_copy(rows_v, out_hbm.at[pl.ds(base, b_per_w)])

    return gather_kernel(table, idx)


def _session_matmul(session_adj, x_item):
    m, k = session_adj.shape
    d = x_item.shape[1]
    bm = 256

    def body(adj_ref, x_ref, out_ref):
        out_ref[...] = jnp.dot(adj_ref[...], x_ref[...],
                               preferred_element_type=jnp.float32)

    return pl.pallas_call(
        body,
        grid=(m // bm,),
        in_specs=[
            pl.BlockSpec((bm, k), lambda i: (i, 0)),
            pl.BlockSpec((k, d), lambda i: (0, 0)),
        ],
        out_specs=pl.BlockSpec((bm, d), lambda i: (i, 0)),
        out_shape=jax.ShapeDtypeStruct((m, d), jnp.float32),
    )(session_adj, x_item)


def _gcn_layer(a, h_in, w, b, relu):
    n = a.shape[0]
    d_in = h_in.shape[1]
    d_out = w.shape[1]
    bm = 256
    b2d = b.reshape(1, d_out)

    def body(h_ref, w_ref, b_ref, a_ref, out_ref, g_scr):
        i = pl.program_id(0)

        @pl.when(i == 0)
        def _():
            g_scr[...] = jnp.dot(h_ref[...], w_ref[...],
                                 preferred_element_type=jnp.float32)

        acc = jnp.dot(a_ref[...], g_scr[...],
                      preferred_element_type=jnp.float32) + b_ref[...]
        out_ref[...] = jnp.maximum(acc, 0.0) if relu else acc

    return pl.pallas_call(
        body,
        grid=(n // bm,),
        in_specs=[
            pl.BlockSpec((n, d_in), lambda i: (0, 0)),
            pl.BlockSpec((d_in, d_out), lambda i: (0, 0)),
            pl.BlockSpec((1, d_out), lambda i: (0, 0)),
            pl.BlockSpec((bm, n), lambda i: (i, 0)),
        ],
        out_specs=pl.BlockSpec((bm, d_out), lambda i: (i, 0)),
        out_shape=jax.ShapeDtypeStruct((n, d_out), jnp.float32),
        scratch_shapes=[pltpu.VMEM((n, d_out), jnp.float32)],
    )(h_in, w, b2d, a)


def _score_matmul(p, q):
    m, d = p.shape
    n = q.shape[0]
    bn = 1024

    def body(p_ref, q_ref, out_ref):
        out_ref[...] = lax.dot_general(
            p_ref[...], q_ref[...],
            (((1,), (1,)), ((), ())),
            preferred_element_type=jnp.float32)

    return pl.pallas_call(
        body,
        grid=(n // bn,),
        in_specs=[
            pl.BlockSpec((m, d), lambda j: (0, 0)),
            pl.BlockSpec((bn, d), lambda j: (j, 0)),
        ],
        out_specs=pl.BlockSpec((m, bn), lambda j: (0, j)),
        out_shape=jax.ShapeDtypeStruct((m, n), jnp.float32),
    )(p, q)


def kernel(batch_idxes, A, item_idxes, session_adj, item_emb_idxes, item_emb,
           W1, b1, W2, b2, W3, b3):
    h3_w = 128
    w3p = jnp.zeros((W3.shape[0], h3_w), W3.dtype).at[:, :W3.shape[1]].set(W3)
    b3p = jnp.zeros((h3_w,), b3.dtype).at[:b3.shape[0]].set(b3)

    x_item = _sc_gather(item_emb, item_emb_idxes)
    x_session = _session_matmul(session_adj, x_item)
    x = jnp.concatenate([x_session, x_item], axis=0)
    h1 = _gcn_layer(A, x, W1, b1, relu=True)
    h2 = _gcn_layer(A, h1, W2, b2, relu=True)
    h3 = _gcn_layer(A, h2, w3p, b3p, relu=False)
    p = _sc_gather(h3, batch_idxes)
    q = _sc_gather(h3, item_idxes)
    return _score_matmul(p, q)
